# baseline (speedup 1.0000x reference)
import jax
import jax.numpy as jnp
from jax import lax
from jax.experimental import pallas as pl
from jax.experimental.pallas import tpu as pltpu
from jax.experimental.pallas import tpu_sc as plsc

NUM_FEATURES = 128
NUM_SAMPLES = 100000
BATCH = 1024
TEMP = 0.05

_SC_ROWS = 64000
_TC_ROWS = NUM_SAMPLES - _SC_ROWS

_NC = 2
_NS = 16
_NW = _NC * _NS
_LANES = 16
_NVEC = NUM_FEATURES // _LANES
_ROWS_W = _SC_ROWS // _NW
_CHUNK = 125
_NCHUNK = _ROWS_W // _CHUNK
_NBUF = 4
_UNROLL = 5

_CHUNK_E = _CHUNK * NUM_FEATURES
_SLAB_E = _ROWS_W * NUM_FEATURES


def _sc_colsum_body(feat_hbm, out_hbm, buf, accv, *sems):
    wid = lax.axis_index("s") * _NC + lax.axis_index("c")
    base = wid * _SLAB_E

    def _copy(chunk_idx, b):
        return pltpu.make_async_copy(
            feat_hbm.at[pl.ds(base + chunk_idx * _CHUNK_E, _CHUNK_E)],
            buf.at[pl.ds(b * _CHUNK_E, _CHUNK_E)],
            sems[b],
        )

    for b in range(_NBUF):
        _copy(b, b).start()

    def _accum_buf(b, acc):
        @plsc.parallel_loop(0, _CHUNK, step=_UNROLL, unroll=5, carry=tuple(acc))
        def _rows(r, acc):
            for u in range(_UNROLL):
                off = b * _CHUNK_E + (r + u) * NUM_FEATURES
                acc = tuple(acc[k] + buf[pl.ds(off + k * _LANES, _LANES)]
                            for k in range(_NVEC))
            return acc
        return list(_rows)

    def _outer(o, acc):
        for b in range(_NBUF):
            g = o * _NBUF + b
            _copy(g, b).wait()
            acc = _accum_buf(b, acc)
            _copy(g + _NBUF, b).start()
        return acc

    acc = [jnp.zeros((_LANES,), jnp.float32) for _ in range(_NVEC)]
    acc = lax.fori_loop(0, _NCHUNK // _NBUF - 1, _outer, acc)
    for b in range(_NBUF):
        _copy(_NCHUNK - _NBUF + b, b).wait()
        acc = _accum_buf(b, acc)

    for k in range(_NVEC):
        accv[pl.ds(k * _LANES, _LANES)] = acc[k]
    pltpu.sync_copy(accv, out_hbm.at[pl.ds(wid * NUM_FEATURES, NUM_FEATURES)])


def _sc_colsum(features_flat):
    run = pl.kernel(
        _sc_colsum_body,
        out_type=jax.ShapeDtypeStruct((_NW * NUM_FEATURES,), jnp.float32),
        mesh=plsc.VectorSubcoreMesh(
            core_axis_name="c", subcore_axis_name="s",
            num_cores=_NC, num_subcores=_NS,
        ),
        scratch_types=[
            pltpu.VMEM((_NBUF * _CHUNK_E,), jnp.float32),
            pltpu.VMEM((NUM_FEATURES,), jnp.float32),
        ] + [pltpu.SemaphoreType.DMA] * _NBUF,
    )
    return run(features_flat).reshape(_NW, NUM_FEATURES)


_TC_BLK = 2000
assert _TC_ROWS % _TC_BLK == 0 and _SC_ROWS % _TC_BLK == 0


def _tc_tail_body(feat_ref, in_ref, mask_ref, colsum_ref, contras_ref):
    i = pl.program_id(0)

    @pl.when(i == 0)
    def _init():
        colsum_ref[...] = jnp.zeros_like(colsum_ref)

    colsum_ref[...] += feat_ref[...].sum(axis=0, keepdims=True)

    @pl.when(i == pl.num_programs(0) - 1)
    def _finish():
        x = in_ref[...]
        m = mask_ref[...]
        xn = x / jnp.sqrt((x * x).sum(axis=1, keepdims=True))
        mn = m / jnp.sqrt((m * m).sum(axis=1, keepdims=True))
        contras = -(xn * mn).sum() / BATCH
        contras_ref[...] = jnp.broadcast_to(contras, contras_ref.shape)


def _tc_tail(features, inputs, mask_inputs_full):
    nblk = _TC_ROWS // _TC_BLK
    first = _SC_ROWS // _TC_BLK
    colsum, contras = pl.pallas_call(
        _tc_tail_body,
        grid=(nblk,),
        in_specs=[
            pl.BlockSpec((_TC_BLK, NUM_FEATURES), lambda i: (i + first, 0)),
            pl.BlockSpec((BATCH, NUM_FEATURES), lambda i: (0, 0)),
            pl.BlockSpec((BATCH, NUM_FEATURES), lambda i: (0, 0)),
        ],
        out_specs=[
            pl.BlockSpec((1, NUM_FEATURES), lambda i: (0, 0)),
            pl.BlockSpec((1, 1), lambda i: (0, 0)),
        ],
        out_shape=[
            jax.ShapeDtypeStruct((1, NUM_FEATURES), jnp.float32),
            jax.ShapeDtypeStruct((1, 1), jnp.float32),
        ],
    )(features, inputs, mask_inputs_full)
    return colsum, contras


def _tc_final_body(part_ref, tail_ref, in_ref, focal_ref):
    colsum = part_ref[...].sum(axis=0, keepdims=True) + tail_ref[...]
    x = in_ref[...]
    s = (x * colsum).sum(axis=1, keepdims=True) / (TEMP * NUM_SAMPLES)
    e = jnp.exp(s)
    p = e / (e + 1e-6)
    loss = -((1.0 - p) ** 2) * jnp.log(p + 1e-6)
    focal = loss.sum() / BATCH
    focal_ref[...] = jnp.broadcast_to(focal, focal_ref.shape)


def _tc_final(sc_partials, tail_colsum, inputs):
    focal = pl.pallas_call(
        _tc_final_body,
        out_shape=jax.ShapeDtypeStruct((1, 1), jnp.float32),
    )(sc_partials, tail_colsum, inputs)
    return focal[0, 0]


def kernel(inputs, mask_inputs_full, indexes, features, labels, labels2, epoch, back):
    sc_partials = _sc_colsum(features.reshape(-1))
    tail_colsum, contras = _tc_tail(features, inputs, mask_inputs_full)
    focal = _tc_final(sc_partials, tail_colsum, inputs)
    contras = contras[0, 0]
    back = jnp.asarray(back)
    return jnp.where(
        jnp.logical_or(back == 1, back == 2), focal + contras * 0.25, focal
    )


# speedup vs baseline: 1.0194x; 1.0194x over previous
import jax
import jax.numpy as jnp
from jax import lax
from jax.experimental import pallas as pl
from jax.experimental.pallas import tpu as pltpu
from jax.experimental.pallas import tpu_sc as plsc

NUM_FEATURES = 128
NUM_SAMPLES = 100000
BATCH = 1024
TEMP = 0.05

_SC_ROWS = 64000
_TC_ROWS = NUM_SAMPLES - _SC_ROWS

_NC = 2
_NS = 16
_NW = _NC * _NS
_LANES = 16
_NVEC = NUM_FEATURES // _LANES
_ROWS_W = _SC_ROWS // _NW
_CHUNK = 125
_NCHUNK = _ROWS_W // _CHUNK
_NBUF = 4
_UNROLL = 5

_CHUNK_E = _CHUNK * NUM_FEATURES
_SLAB_E = _ROWS_W * NUM_FEATURES


def _sc_colsum_body(feat_hbm, out_hbm, buf, accv, *sems):
    wid = lax.axis_index("s") * _NC + lax.axis_index("c")
    base = wid * _SLAB_E

    def _copy(chunk_idx, b):
        return pltpu.make_async_copy(
            feat_hbm.at[pl.ds(base + chunk_idx * _CHUNK_E, _CHUNK_E)],
            buf.at[pl.ds(b * _CHUNK_E, _CHUNK_E)],
            sems[b],
        )

    for b in range(_NBUF):
        _copy(b, b).start()

    def _accum_buf(b, acc):
        @plsc.parallel_loop(0, _CHUNK, step=_UNROLL, unroll=5, carry=tuple(acc))
        def _rows(r, acc):
            for u in range(_UNROLL):
                off = b * _CHUNK_E + (r + u) * NUM_FEATURES
                acc = tuple(acc[k] + buf[pl.ds(off + k * _LANES, _LANES)]
                            for k in range(_NVEC))
            return acc
        return list(_rows)

    def _outer(o, acc):
        for b in range(_NBUF):
            g = o * _NBUF + b
            _copy(g, b).wait()
            acc = _accum_buf(b, acc)
            _copy(g + _NBUF, b).start()
        return acc

    acc = [jnp.zeros((_LANES,), jnp.float32) for _ in range(_NVEC)]
    acc = lax.fori_loop(0, _NCHUNK // _NBUF - 1, _outer, acc)
    for b in range(_NBUF):
        _copy(_NCHUNK - _NBUF + b, b).wait()
        acc = _accum_buf(b, acc)

    for k in range(_NVEC):
        accv[pl.ds(k * _LANES, _LANES)] = acc[k]
    pltpu.sync_copy(accv, out_hbm.at[pl.ds(wid * NUM_FEATURES, NUM_FEATURES)])


def _sc_colsum(features_flat):
    run = pl.kernel(
        _sc_colsum_body,
        out_type=jax.ShapeDtypeStruct((_NW * NUM_FEATURES,), jnp.float32),
        mesh=plsc.VectorSubcoreMesh(
            core_axis_name="c", subcore_axis_name="s",
            num_cores=_NC, num_subcores=_NS,
        ),
        scratch_types=[
            pltpu.VMEM((_NBUF * _CHUNK_E,), jnp.float32),
            pltpu.VMEM((NUM_FEATURES,), jnp.float32),
        ] + [pltpu.SemaphoreType.DMA] * _NBUF,
    )
    return run(features_flat).reshape(_NW, NUM_FEATURES)


_TC_BLK = 4000
assert _TC_ROWS % _TC_BLK == 0 and _SC_ROWS % _TC_BLK == 0


def _tc_tail_body(feat_ref, in_ref, mask_ref, colsum_ref, contras_ref):
    i = pl.program_id(0)

    @pl.when(i == 0)
    def _init():
        colsum_ref[...] = jnp.zeros_like(colsum_ref)

    colsum_ref[...] += feat_ref[...].sum(axis=0, keepdims=True)

    @pl.when(i == pl.num_programs(0) - 1)
    def _finish():
        x = in_ref[...]
        m = mask_ref[...]
        xn = x / jnp.sqrt((x * x).sum(axis=1, keepdims=True))
        mn = m / jnp.sqrt((m * m).sum(axis=1, keepdims=True))
        contras = -(xn * mn).sum() / BATCH
        contras_ref[...] = jnp.broadcast_to(contras, contras_ref.shape)


def _tc_tail(features, inputs, mask_inputs_full):
    nblk = _TC_ROWS // _TC_BLK
    first = _SC_ROWS // _TC_BLK
    colsum, contras = pl.pallas_call(
        _tc_tail_body,
        grid=(nblk,),
        in_specs=[
            pl.BlockSpec((_TC_BLK, NUM_FEATURES), lambda i: (i + first, 0)),
            pl.BlockSpec((BATCH, NUM_FEATURES), lambda i: (0, 0)),
            pl.BlockSpec((BATCH, NUM_FEATURES), lambda i: (0, 0)),
        ],
        out_specs=[
            pl.BlockSpec((1, NUM_FEATURES), lambda i: (0, 0)),
            pl.BlockSpec((1, 1), lambda i: (0, 0)),
        ],
        out_shape=[
            jax.ShapeDtypeStruct((1, NUM_FEATURES), jnp.float32),
            jax.ShapeDtypeStruct((1, 1), jnp.float32),
        ],
    )(features, inputs, mask_inputs_full)
    return colsum, contras


def _tc_final_body(part_ref, tail_ref, in_ref, focal_ref):
    colsum = part_ref[...].sum(axis=0, keepdims=True) + tail_ref[...]
    x = in_ref[...]
    s = (x * colsum).sum(axis=1, keepdims=True) / (TEMP * NUM_SAMPLES)
    e = jnp.exp(s)
    p = e / (e + 1e-6)
    loss = -((1.0 - p) ** 2) * jnp.log(p + 1e-6)
    focal = loss.sum() / BATCH
    focal_ref[...] = jnp.broadcast_to(focal, focal_ref.shape)


def _tc_final(sc_partials, tail_colsum, inputs):
    focal = pl.pallas_call(
        _tc_final_body,
        out_shape=jax.ShapeDtypeStruct((1, 1), jnp.float32),
    )(sc_partials, tail_colsum, inputs)
    return focal[0, 0]


def kernel(inputs, mask_inputs_full, indexes, features, labels, labels2, epoch, back):
    sc_partials = _sc_colsum(features.reshape(-1))
    tail_colsum, contras = _tc_tail(features, inputs, mask_inputs_full)
    focal = _tc_final(sc_partials, tail_colsum, inputs)
    contras = contras[0, 0]
    back = jnp.asarray(back)
    return jnp.where(
        jnp.logical_or(back == 1, back == 2), focal + contras * 0.25, focal
    )
